# baseline (speedup 1.0000x reference)
import functools
import jax
import jax.numpy as jnp
from jax import lax
from jax.experimental import pallas as pl
from jax.experimental.pallas import tpu as pltpu
from jax.experimental.pallas import tpu_sc as plsc

N_PTS = 1048576
FDIM = 16
G = 128
NC, NS, L = 2, 16, 16
NW = NC * NS
PTS_PER_W = N_PTS // NW
B = 128
NBLK = PTS_PER_W // B
BRICK = 8 * FDIM


def _axis_terms(v):
    xn = (v + 1.0) - 1.0
    ip = ((xn + 1.0) * float(G) - 1.0) * 0.5
    i0 = ip.astype(jnp.int32)
    i0 = jnp.where(i0.astype(jnp.float32) > ip, i0 - 1, i0)
    w = ip - i0.astype(jnp.float32)
    i1 = i0 + 1
    ok0 = (i0 >= 0) & (i0 < G)
    ok1 = (i1 >= 0) & (i1 < G)
    w0 = jnp.where(ok0, 1.0 - w, 0.0)
    w1 = jnp.where(ok1, w, 0.0)
    i0c = jnp.minimum(jnp.maximum(i0, 0), G - 1)
    i1c = jnp.minimum(jnp.maximum(i1, 0), G - 1)
    base = jnp.minimum(jnp.maximum(i0, 0), G - 2)
    we0 = jnp.where(i0c == base, w0, 0.0) + jnp.where(i1c == base, w1, 0.0)
    we1 = (jnp.where(i0c == base + 1, w0, 0.0)
           + jnp.where(i1c == base + 1, w1, 0.0))
    return base, we0, we1


def _sc_body(table, xq, yq, zq, out, cb, idxb, wb, rows, ob,
             sem_c, sem_g, sem_o):
    wid = lax.axis_index("s") * NC + lax.axis_index("c")
    base0 = wid * PTS_PER_W
    coords = (xq, yq, zq)

    def start_coords(i, p):
        for a in range(3):
            pltpu.async_copy(
                coords[a].at[pl.ds(base0 + i * B, B)], cb.at[p, a], sem_c)

    def drain_coords(p):
        for a in range(3):
            pltpu.make_async_copy(
                coords[a].at[pl.ds(0, B)], cb.at[p, a], sem_c).wait()

    def compute_idx(p):
        for j in range(B // L):
            sl = pl.ds(j * L, L)
            xb, wx0, wx1 = _axis_terms(cb[p, 0, sl])
            yb, wy0, wy1 = _axis_terms(cb[p, 1, sl])
            zb, wz0, wz1 = _axis_terms(cb[p, 2, sl])
            idxb[p, sl] = zb * (G * G) + yb * G + xb
            pxy = (wx0 * wy0, wx1 * wy0, wx0 * wy1, wx1 * wy1)
            wzs = (wz0, wz1)
            for ez in (0, 1):
                for ey in (0, 1):
                    for ex in (0, 1):
                        e = ez * 4 + ey * 2 + ex
                        wb[p, e, sl] = pxy[ey * 2 + ex] * wzs[ez]

    def fire_gathers(p):
        pltpu.async_copy(table.at[idxb.at[p]], rows.at[p], sem_g)

    def drain_gathers(p):
        pltpu.make_async_copy(table.at[idxb.at[p]], rows.at[p], sem_g).wait()

    def accumulate(p):
        def acc(g, carry2):
            sl = pl.ds(g * L, L)
            wv = [wb[p, e, sl] for e in range(8)]
            for k in range(L):
                b = g * L + k
                a = wv[0][k] * rows[p, b, pl.ds(0, L)]
                for e in range(1, 8):
                    a = a + wv[e][k] * rows[p, b, pl.ds(e * L, L)]
                ob[p, b, :] = a
            return carry2

        lax.fori_loop(0, B // L, acc, 0)

    def fire_out(j, p):
        pltpu.async_copy(ob.at[p], out.at[pl.ds(base0 + j * B, B)], sem_o)

    def drain_out(p):
        pltpu.make_async_copy(ob.at[p], out.at[pl.ds(0, B)], sem_o).wait()

    pltpu.sync_copy(xq.at[pl.ds(base0, B)], cb.at[0, 0])
    pltpu.sync_copy(yq.at[pl.ds(base0, B)], cb.at[0, 1])
    pltpu.sync_copy(zq.at[pl.ds(base0, B)], cb.at[0, 2])
    compute_idx(0)
    fire_gathers(0)
    start_coords(1, 1)

    def block(i, carry):
        p = lax.bitwise_and(i, 1)
        q = 1 - p
        drain_coords(p)
        compute_idx(p)
        drain_gathers(q)
        fire_gathers(p)
        @pl.when(i < NBLK - 1)
        def _():
            start_coords(i + 1, q)
        @pl.when(i >= 3)
        def _():
            drain_out(q)
        accumulate(q)
        fire_out(i - 1, q)
        return carry

    lax.fori_loop(1, NBLK, block, 0)

    pl_last = (NBLK - 1) & 1
    drain_gathers(pl_last)
    drain_out(pl_last)
    accumulate(pl_last)
    fire_out(NBLK - 1, pl_last)
    drain_out(1 - pl_last)
    drain_out(pl_last)


@functools.partial(
    pl.kernel,
    out_type=jax.ShapeDtypeStruct((N_PTS, FDIM), jnp.float32),
    mesh=plsc.VectorSubcoreMesh(core_axis_name="c", subcore_axis_name="s"),
    scratch_types=[
        pltpu.VMEM((2, 3, B), jnp.float32),
        pltpu.VMEM((2, B), jnp.int32),
        pltpu.VMEM((2, 8, B), jnp.float32),
        pltpu.VMEM((2, B, BRICK), jnp.float32),
        pltpu.VMEM((2, B, FDIM), jnp.float32),
        pltpu.SemaphoreType.DMA,
        pltpu.SemaphoreType.DMA,
        pltpu.SemaphoreType.DMA,
    ],
    compiler_params=pltpu.CompilerParams(
        use_tc_tiling_on_sc=False, needs_layout_passes=False
    ),
)
def _grid_sample_sc(table, xq, yq, zq, out, cb, idxb, wb, rows, ob,
                    sem_c, sem_g, sem_o):
    _sc_body(table, xq, yq, zq, out, cb, idxb, wb, rows, ob,
             sem_c, sem_g, sem_o)


def kernel(x, feature):
    table = jnp.transpose(feature[0], (1, 2, 3, 0)).reshape(G * G * G, FDIM)
    parts = []
    for dz in (0, 1):
        for dy in (0, 1):
            for dx in (0, 1):
                off = dz * (G * G) + dy * G + dx
                parts.append(table if off == 0 else
                             jnp.concatenate([table[off:], table[:off]],
                                             axis=0))
    nbr = jnp.concatenate(parts, axis=1)
    xq = x[:, 0]
    yq = x[:, 1]
    zq = x[:, 2]
    return _grid_sample_sc(nbr, xq, yq, zq)


# speedup vs baseline: 8.7177x; 8.7177x over previous
import functools
import jax
import jax.numpy as jnp
from jax import lax
from jax.experimental import pallas as pl
from jax.experimental.pallas import tpu as pltpu
from jax.experimental.pallas import tpu_sc as plsc

N_PTS = 1048576
FDIM = 16
G = 128
NC, NS, L = 2, 16, 16
NW = NC * NS
PTS_PER_W = N_PTS // NW
B = 128
NBLK = PTS_PER_W // B


def _axis_terms(v):
    xn = (v + 1.0) - 1.0
    ip = ((xn + 1.0) * float(G) - 1.0) * 0.5
    i0 = ip.astype(jnp.int32)
    i0 = jnp.where(i0.astype(jnp.float32) > ip, i0 - 1, i0)
    w = ip - i0.astype(jnp.float32)
    i1 = i0 + 1
    ok0 = (i0 >= 0) & (i0 < G)
    ok1 = (i1 >= 0) & (i1 < G)
    w0 = jnp.where(ok0, 1.0 - w, 0.0)
    w1 = jnp.where(ok1, w, 0.0)
    i0c = jnp.minimum(jnp.maximum(i0, 0), G - 1)
    i1c = jnp.minimum(jnp.maximum(i1, 0), G - 1)
    return i0c, i1c, w0, w1


def _sc_body(table, xq, yq, zq, out, cb, idxb, wb, rows, ob,
             sem_c, sem_g, sem_o):
    wid = lax.axis_index("s") * NC + lax.axis_index("c")
    base0 = wid * PTS_PER_W
    coords = (xq, yq, zq)

    def start_coords(i, p):
        for a in range(3):
            pltpu.async_copy(
                coords[a].at[pl.ds(base0 + i * B, B)], cb.at[p, a], sem_c)

    def drain_coords(p):
        for a in range(3):
            pltpu.make_async_copy(
                coords[a].at[pl.ds(0, B)], cb.at[p, a], sem_c).wait()

    def compute_idx(p):
        for j in range(B // L):
            sl = pl.ds(j * L, L)
            x0, x1, wx0, wx1 = _axis_terms(cb[p, 0, sl])
            y0, y1, wy0, wy1 = _axis_terms(cb[p, 1, sl])
            z0, z1, wz0, wz1 = _axis_terms(cb[p, 2, sl])
            pxy = (wx0 * wy0, wx1 * wy0, wx0 * wy1, wx1 * wy1)
            xs = (x0, x1)
            ys = (y0 * G, y1 * G)
            zs = (z0 * (G * G), z1 * (G * G))
            wzs = (wz0, wz1)
            for cz in (0, 1):
                for cy in (0, 1):
                    for cx in (0, 1):
                        c = cz * 4 + cy * 2 + cx
                        idxb[p, c, sl] = zs[cz] + ys[cy] + xs[cx]
                        wb[p, c, sl] = pxy[cy * 2 + cx] * wzs[cz]

    BH = B // 2

    def fire_gathers(p):
        for c in range(8):
            for h in range(2):
                sl = pl.ds(h * BH, BH)
                pltpu.async_copy(
                    table.at[idxb.at[p, c, sl]], rows.at[p, c, sl], sem_g)

    def drain_gathers(p):
        for c in range(8):
            for h in range(2):
                sl = pl.ds(h * BH, BH)
                pltpu.make_async_copy(
                    table.at[idxb.at[p, c, sl]], rows.at[p, c, sl],
                    sem_g).wait()

    def accumulate(p):
        def acc(g, carry2):
            sl = pl.ds(g * L, L)
            wv = [wb[p, c, sl] for c in range(8)]
            for k in range(L):
                b = g * L + k
                a = wv[0][k] * rows[p, 0, b, :]
                for c in range(1, 8):
                    a = a + wv[c][k] * rows[p, c, b, :]
                ob[p, b, :] = a
            return carry2

        lax.fori_loop(0, B // L, acc, 0)

    def fire_out(j, p):
        pltpu.async_copy(ob.at[p], out.at[pl.ds(base0 + j * B, B)], sem_o)

    def drain_out(p):
        pltpu.make_async_copy(ob.at[p], out.at[pl.ds(0, B)], sem_o).wait()

    pltpu.sync_copy(xq.at[pl.ds(base0, B)], cb.at[0, 0])
    pltpu.sync_copy(yq.at[pl.ds(base0, B)], cb.at[0, 1])
    pltpu.sync_copy(zq.at[pl.ds(base0, B)], cb.at[0, 2])
    compute_idx(0)
    fire_gathers(0)
    start_coords(1, 1)

    def block(i, carry):
        p = lax.bitwise_and(i, 1)
        q = 1 - p
        drain_coords(p)
        compute_idx(p)
        drain_gathers(q)
        fire_gathers(p)
        @pl.when(i < NBLK - 1)
        def _():
            start_coords(i + 1, q)
        @pl.when(i >= 3)
        def _():
            drain_out(q)
        accumulate(q)
        fire_out(i - 1, q)
        return carry

    lax.fori_loop(1, NBLK, block, 0)

    pl_last = (NBLK - 1) & 1
    drain_gathers(pl_last)
    drain_out(pl_last)
    accumulate(pl_last)
    fire_out(NBLK - 1, pl_last)
    drain_out(1 - pl_last)
    drain_out(pl_last)


@functools.partial(
    pl.kernel,
    out_type=jax.ShapeDtypeStruct((N_PTS, FDIM), jnp.float32),
    mesh=plsc.VectorSubcoreMesh(core_axis_name="c", subcore_axis_name="s"),
    scratch_types=[
        pltpu.VMEM((2, 3, B), jnp.float32),
        pltpu.VMEM((2, 8, B), jnp.int32),
        pltpu.VMEM((2, 8, B), jnp.float32),
        pltpu.VMEM((2, 8, B, FDIM), jnp.float32),
        pltpu.VMEM((2, B, FDIM), jnp.float32),
        pltpu.SemaphoreType.DMA,
        pltpu.SemaphoreType.DMA,
        pltpu.SemaphoreType.DMA,
    ],
    compiler_params=pltpu.CompilerParams(
        use_tc_tiling_on_sc=False, needs_layout_passes=False
    ),
)
def _grid_sample_sc(table, xq, yq, zq, out, cb, idxb, wb, rows, ob,
                    sem_c, sem_g, sem_o):
    _sc_body(table, xq, yq, zq, out, cb, idxb, wb, rows, ob,
             sem_c, sem_g, sem_o)


def kernel(x, feature):
    table = jnp.transpose(feature[0], (1, 2, 3, 0)).reshape(G * G * G, FDIM)
    xq = x[:, 0]
    yq = x[:, 1]
    zq = x[:, 2]
    return _grid_sample_sc(table, xq, yq, zq)


# speedup vs baseline: 8.7387x; 1.0024x over previous
import functools
import jax
import jax.numpy as jnp
from jax import lax
from jax.experimental import pallas as pl
from jax.experimental.pallas import tpu as pltpu
from jax.experimental.pallas import tpu_sc as plsc

N_PTS = 1048576
FDIM = 16
G = 128
NC, NS, L = 2, 16, 16
NW = NC * NS
PTS_PER_W = N_PTS // NW
B = 256
NBLK = PTS_PER_W // B


def _axis_terms(v):
    xn = (v + 1.0) - 1.0
    ip = ((xn + 1.0) * float(G) - 1.0) * 0.5
    i0 = ip.astype(jnp.int32)
    i0 = jnp.where(i0.astype(jnp.float32) > ip, i0 - 1, i0)
    w = ip - i0.astype(jnp.float32)
    i1 = i0 + 1
    ok0 = (i0 >= 0) & (i0 < G)
    ok1 = (i1 >= 0) & (i1 < G)
    w0 = jnp.where(ok0, 1.0 - w, 0.0)
    w1 = jnp.where(ok1, w, 0.0)
    i0c = jnp.minimum(jnp.maximum(i0, 0), G - 1)
    i1c = jnp.minimum(jnp.maximum(i1, 0), G - 1)
    return i0c, i1c, w0, w1


def _sc_body(table, cq, out, cb, idxb, wb, rows, ob, sem_c, sem_g, sem_o):
    wid = lax.axis_index("s") * NC + lax.axis_index("c")
    base0 = wid * PTS_PER_W
    blk0 = wid * NBLK

    def start_coords(i, p):
        pltpu.async_copy(
            cq.at[pl.ds((blk0 + i) * 3 * B, 3 * B)], cb.at[p], sem_c)

    def drain_coords(p):
        pltpu.make_async_copy(
            cq.at[pl.ds(0, 3 * B)], cb.at[p], sem_c).wait()

    def compute_idx(p):
        for j in range(B // L):
            sl = pl.ds(j * L, L)
            x0, x1, wx0, wx1 = _axis_terms(cb[p, pl.ds(0 * B + j * L, L)])
            y0, y1, wy0, wy1 = _axis_terms(cb[p, pl.ds(1 * B + j * L, L)])
            z0, z1, wz0, wz1 = _axis_terms(cb[p, pl.ds(2 * B + j * L, L)])
            pxy = (wx0 * wy0, wx1 * wy0, wx0 * wy1, wx1 * wy1)
            xs = (x0, x1)
            ys = (y0 * G, y1 * G)
            zs = (z0 * (G * G), z1 * (G * G))
            wzs = (wz0, wz1)
            for cz in (0, 1):
                for cy in (0, 1):
                    for cx in (0, 1):
                        c = cz * 4 + cy * 2 + cx
                        idxb[p, c, sl] = zs[cz] + ys[cy] + xs[cx]
                        wb[p, c, sl] = pxy[cy * 2 + cx] * wzs[cz]

    BH = B // 2

    def fire_gathers(p):
        for c in range(8):
            for h in range(2):
                sl = pl.ds(h * BH, BH)
                pltpu.async_copy(
                    table.at[idxb.at[p, c, sl]], rows.at[p, c, sl], sem_g)

    def drain_gathers(p):
        for c in range(8):
            for h in range(2):
                sl = pl.ds(h * BH, BH)
                pltpu.make_async_copy(
                    table.at[idxb.at[p, c, sl]], rows.at[p, c, sl],
                    sem_g).wait()

    def accumulate(p):
        def acc(g, carry2):
            sl = pl.ds(g * L, L)
            wv = [wb[p, c, sl] for c in range(8)]
            for k in range(L):
                b = g * L + k
                a = wv[0][k] * rows[p, 0, b, :]
                for c in range(1, 8):
                    a = a + wv[c][k] * rows[p, c, b, :]
                ob[p, b, :] = a
            return carry2

        lax.fori_loop(0, B // L, acc, 0)

    def fire_out(j, p):
        pltpu.async_copy(ob.at[p], out.at[pl.ds(base0 + j * B, B)], sem_o)

    def drain_out(p):
        pltpu.make_async_copy(ob.at[p], out.at[pl.ds(0, B)], sem_o).wait()

    pltpu.sync_copy(cq.at[pl.ds(blk0 * 3 * B, 3 * B)], cb.at[0])
    compute_idx(0)
    fire_gathers(0)
    start_coords(1, 1)

    def block(i, carry):
        p = lax.bitwise_and(i, 1)
        q = 1 - p
        drain_coords(p)
        compute_idx(p)
        drain_gathers(q)
        fire_gathers(p)
        @pl.when(i < NBLK - 1)
        def _():
            start_coords(i + 1, q)
        @pl.when(i >= 3)
        def _():
            drain_out(q)
        accumulate(q)
        fire_out(i - 1, q)
        return carry

    lax.fori_loop(1, NBLK, block, 0)

    pl_last = (NBLK - 1) & 1
    drain_gathers(pl_last)
    drain_out(pl_last)
    accumulate(pl_last)
    fire_out(NBLK - 1, pl_last)
    drain_out(1 - pl_last)
    drain_out(pl_last)


@functools.partial(
    pl.kernel,
    out_type=jax.ShapeDtypeStruct((N_PTS, FDIM), jnp.float32),
    mesh=plsc.VectorSubcoreMesh(core_axis_name="c", subcore_axis_name="s"),
    scratch_types=[
        pltpu.VMEM((2, 3 * B), jnp.float32),
        pltpu.VMEM((2, 8, B), jnp.int32),
        pltpu.VMEM((2, 8, B), jnp.float32),
        pltpu.VMEM((2, 8, B, FDIM), jnp.float32),
        pltpu.VMEM((2, B, FDIM), jnp.float32),
        pltpu.SemaphoreType.DMA,
        pltpu.SemaphoreType.DMA,
        pltpu.SemaphoreType.DMA,
    ],
    compiler_params=pltpu.CompilerParams(
        use_tc_tiling_on_sc=False, needs_layout_passes=False
    ),
)
def _grid_sample_sc(table, cq, out, cb, idxb, wb, rows, ob,
                    sem_c, sem_g, sem_o):
    _sc_body(table, cq, out, cb, idxb, wb, rows, ob, sem_c, sem_g, sem_o)


def kernel(x, feature):
    table = jnp.transpose(feature[0], (1, 2, 3, 0)).reshape(G * G * G, FDIM)
    cq = jnp.stack(
        [x[:, 0].reshape(-1, B), x[:, 1].reshape(-1, B),
         x[:, 2].reshape(-1, B)], axis=1).reshape(-1)
    return _grid_sample_sc(table, cq)


# speedup vs baseline: 9.6128x; 1.1000x over previous
import functools
import jax
import jax.numpy as jnp
from jax import lax
from jax.experimental import pallas as pl
from jax.experimental.pallas import tpu as pltpu
from jax.experimental.pallas import tpu_sc as plsc

N_PTS = 1048576
H_PTS = N_PTS // 2
FDIM = 16
G = 128
NC, NS, L = 2, 16, 16
NW = NC * NS
PTS_PER_W = H_PTS // NW
B = 256
NBLK = PTS_PER_W // B


def _axis_terms(v):
    xn = (v + 1.0) - 1.0
    ip = ((xn + 1.0) * float(G) - 1.0) * 0.5
    i0 = ip.astype(jnp.int32)
    i0 = jnp.where(i0.astype(jnp.float32) > ip, i0 - 1, i0)
    w = ip - i0.astype(jnp.float32)
    i1 = i0 + 1
    ok0 = (i0 >= 0) & (i0 < G)
    ok1 = (i1 >= 0) & (i1 < G)
    w0 = jnp.where(ok0, 1.0 - w, 0.0)
    w1 = jnp.where(ok1, w, 0.0)
    i0c = jnp.minimum(jnp.maximum(i0, 0), G - 1)
    i1c = jnp.minimum(jnp.maximum(i1, 0), G - 1)
    return i0c, i1c, w0, w1


def _sc_body(table, cq, out, cb, idxb, wb, rows, ob, sem_c, sem_g, sem_o):
    wid = lax.axis_index("s") * NC + lax.axis_index("c")
    base0 = wid * PTS_PER_W
    blk0 = wid * NBLK

    def start_coords(i, p):
        pltpu.async_copy(
            cq.at[pl.ds((blk0 + i) * 3 * B, 3 * B)], cb.at[p], sem_c)

    def drain_coords(p):
        pltpu.make_async_copy(
            cq.at[pl.ds(0, 3 * B)], cb.at[p], sem_c).wait()

    def compute_idx(p):
        for j in range(B // L):
            sl = pl.ds(j * L, L)
            x0, x1, wx0, wx1 = _axis_terms(cb[p, pl.ds(0 * B + j * L, L)])
            y0, y1, wy0, wy1 = _axis_terms(cb[p, pl.ds(1 * B + j * L, L)])
            z0, z1, wz0, wz1 = _axis_terms(cb[p, pl.ds(2 * B + j * L, L)])
            pxy = (wx0 * wy0, wx1 * wy0, wx0 * wy1, wx1 * wy1)
            xs = (x0, x1)
            ys = (y0 * G, y1 * G)
            zs = (z0 * (G * G), z1 * (G * G))
            wzs = (wz0, wz1)
            for cz in (0, 1):
                for cy in (0, 1):
                    for cx in (0, 1):
                        c = cz * 4 + cy * 2 + cx
                        idxb[p, c, sl] = zs[cz] + ys[cy] + xs[cx]
                        wb[p, c, sl] = pxy[cy * 2 + cx] * wzs[cz]

    BH = B // 2

    def fire_gathers(p):
        for c in range(8):
            for h in range(2):
                sl = pl.ds(h * BH, BH)
                pltpu.async_copy(
                    table.at[idxb.at[p, c, sl]], rows.at[p, c, sl], sem_g)

    def drain_gathers(p):
        for c in range(8):
            for h in range(2):
                sl = pl.ds(h * BH, BH)
                pltpu.make_async_copy(
                    table.at[idxb.at[p, c, sl]], rows.at[p, c, sl],
                    sem_g).wait()

    def accumulate(p):
        def acc(g, carry2):
            sl = pl.ds(g * L, L)
            wv = [wb[p, c, sl] for c in range(8)]
            for k in range(L):
                b = g * L + k
                a = wv[0][k] * rows[p, 0, b, :]
                for c in range(1, 8):
                    a = a + wv[c][k] * rows[p, c, b, :]
                ob[p, b, :] = a
            return carry2

        lax.fori_loop(0, B // L, acc, 0)

    def fire_out(j, p):
        pltpu.async_copy(ob.at[p], out.at[pl.ds(base0 + j * B, B)], sem_o)

    def drain_out(p):
        pltpu.make_async_copy(ob.at[p], out.at[pl.ds(0, B)], sem_o).wait()

    pltpu.sync_copy(cq.at[pl.ds(blk0 * 3 * B, 3 * B)], cb.at[0])
    compute_idx(0)
    fire_gathers(0)
    start_coords(1, 1)

    def block(i, carry):
        p = lax.bitwise_and(i, 1)
        q = 1 - p
        drain_coords(p)
        compute_idx(p)
        drain_gathers(q)
        fire_gathers(p)
        @pl.when(i < NBLK - 1)
        def _():
            start_coords(i + 1, q)
        @pl.when(i >= 3)
        def _():
            drain_out(q)
        accumulate(q)
        fire_out(i - 1, q)
        return carry

    lax.fori_loop(1, NBLK, block, 0)

    pl_last = (NBLK - 1) & 1
    drain_gathers(pl_last)
    drain_out(pl_last)
    accumulate(pl_last)
    fire_out(NBLK - 1, pl_last)
    drain_out(1 - pl_last)
    drain_out(pl_last)


@functools.partial(
    pl.kernel,
    out_type=jax.ShapeDtypeStruct((H_PTS, FDIM), jnp.float32),
    mesh=plsc.VectorSubcoreMesh(core_axis_name="c", subcore_axis_name="s"),
    scratch_types=[
        pltpu.VMEM((2, 3 * B), jnp.float32),
        pltpu.VMEM((2, 8, B), jnp.int32),
        pltpu.VMEM((2, 8, B), jnp.float32),
        pltpu.VMEM((2, 8, B, FDIM), jnp.float32),
        pltpu.VMEM((2, B, FDIM), jnp.float32),
        pltpu.SemaphoreType.DMA,
        pltpu.SemaphoreType.DMA,
        pltpu.SemaphoreType.DMA,
    ],
    compiler_params=pltpu.CompilerParams(
        use_tc_tiling_on_sc=False, needs_layout_passes=False
    ),
)
def _grid_sample_sc(table, cq, out, cb, idxb, wb, rows, ob,
                    sem_c, sem_g, sem_o):
    _sc_body(table, cq, out, cb, idxb, wb, rows, ob, sem_c, sem_g, sem_o)


def kernel(x, feature):
    table = jnp.transpose(feature[0], (1, 2, 3, 0)).reshape(G * G * G, FDIM)

    def half(xh):
        cq = jnp.stack(
            [xh[:, 0].reshape(-1, B), xh[:, 1].reshape(-1, B),
             xh[:, 2].reshape(-1, B)], axis=1).reshape(-1)
        return _grid_sample_sc(table, cq)

    return jnp.concatenate([half(x[:H_PTS]), half(x[H_PTS:])], axis=0)


# speedup vs baseline: 9.7318x; 1.0124x over previous
import functools
import jax
import jax.numpy as jnp
from jax import lax
from jax.experimental import pallas as pl
from jax.experimental.pallas import tpu as pltpu
from jax.experimental.pallas import tpu_sc as plsc

N_PTS = 1048576
H_PTS = N_PTS // 4
FDIM = 16
G = 128
NC, NS, L = 2, 16, 16
NW = NC * NS
PTS_PER_W = H_PTS // NW
B = 256
NBLK = PTS_PER_W // B


def _axis_terms(v):
    xn = (v + 1.0) - 1.0
    ip = ((xn + 1.0) * float(G) - 1.0) * 0.5
    i0 = ip.astype(jnp.int32)
    i0 = jnp.where(i0.astype(jnp.float32) > ip, i0 - 1, i0)
    w = ip - i0.astype(jnp.float32)
    i1 = i0 + 1
    ok0 = (i0 >= 0) & (i0 < G)
    ok1 = (i1 >= 0) & (i1 < G)
    w0 = jnp.where(ok0, 1.0 - w, 0.0)
    w1 = jnp.where(ok1, w, 0.0)
    i0c = jnp.minimum(jnp.maximum(i0, 0), G - 1)
    i1c = jnp.minimum(jnp.maximum(i1, 0), G - 1)
    return i0c, i1c, w0, w1


def _sc_body(table, cq, out, cb, idxb, wb, rows, ob, sem_c, sem_g, sem_o):
    wid = lax.axis_index("s") * NC + lax.axis_index("c")
    base0 = wid * PTS_PER_W
    blk0 = wid * NBLK

    def start_coords(i, p):
        pltpu.async_copy(
            cq.at[pl.ds((blk0 + i) * 3 * B, 3 * B)], cb.at[p], sem_c)

    def drain_coords(p):
        pltpu.make_async_copy(
            cq.at[pl.ds(0, 3 * B)], cb.at[p], sem_c).wait()

    def compute_idx(p):
        for j in range(B // L):
            sl = pl.ds(j * L, L)
            x0, x1, wx0, wx1 = _axis_terms(cb[p, pl.ds(0 * B + j * L, L)])
            y0, y1, wy0, wy1 = _axis_terms(cb[p, pl.ds(1 * B + j * L, L)])
            z0, z1, wz0, wz1 = _axis_terms(cb[p, pl.ds(2 * B + j * L, L)])
            pxy = (wx0 * wy0, wx1 * wy0, wx0 * wy1, wx1 * wy1)
            xs = (x0, x1)
            ys = (y0 * G, y1 * G)
            zs = (z0 * (G * G), z1 * (G * G))
            wzs = (wz0, wz1)
            for cz in (0, 1):
                for cy in (0, 1):
                    for cx in (0, 1):
                        c = cz * 4 + cy * 2 + cx
                        idxb[p, c, sl] = zs[cz] + ys[cy] + xs[cx]
                        wb[p, c, sl] = pxy[cy * 2 + cx] * wzs[cz]

    BH = B // 2

    def fire_gathers(p):
        for c in range(8):
            for h in range(2):
                sl = pl.ds(h * BH, BH)
                pltpu.async_copy(
                    table.at[idxb.at[p, c, sl]], rows.at[p, c, sl], sem_g)

    def drain_gathers(p):
        for c in range(8):
            for h in range(2):
                sl = pl.ds(h * BH, BH)
                pltpu.make_async_copy(
                    table.at[idxb.at[p, c, sl]], rows.at[p, c, sl],
                    sem_g).wait()

    def accumulate(p):
        def acc(g, carry2):
            sl = pl.ds(g * L, L)
            wv = [wb[p, c, sl] for c in range(8)]
            for k in range(L):
                b = g * L + k
                a = wv[0][k] * rows[p, 0, b, :]
                for c in range(1, 8):
                    a = a + wv[c][k] * rows[p, c, b, :]
                ob[p, b, :] = a
            return carry2

        lax.fori_loop(0, B // L, acc, 0)

    def fire_out(j, p):
        pltpu.async_copy(ob.at[p], out.at[pl.ds(base0 + j * B, B)], sem_o)

    def drain_out(p):
        pltpu.make_async_copy(ob.at[p], out.at[pl.ds(0, B)], sem_o).wait()

    pltpu.sync_copy(cq.at[pl.ds(blk0 * 3 * B, 3 * B)], cb.at[0])
    compute_idx(0)
    fire_gathers(0)
    start_coords(1, 1)

    def block(i, carry):
        p = lax.bitwise_and(i, 1)
        q = 1 - p
        drain_coords(p)
        compute_idx(p)
        drain_gathers(q)
        fire_gathers(p)
        @pl.when(i < NBLK - 1)
        def _():
            start_coords(i + 1, q)
        @pl.when(i >= 3)
        def _():
            drain_out(q)
        accumulate(q)
        fire_out(i - 1, q)
        return carry

    lax.fori_loop(1, NBLK, block, 0)

    pl_last = (NBLK - 1) & 1
    drain_gathers(pl_last)
    drain_out(pl_last)
    accumulate(pl_last)
    fire_out(NBLK - 1, pl_last)
    drain_out(1 - pl_last)
    drain_out(pl_last)


@functools.partial(
    pl.kernel,
    out_type=jax.ShapeDtypeStruct((H_PTS, FDIM), jnp.float32),
    mesh=plsc.VectorSubcoreMesh(core_axis_name="c", subcore_axis_name="s"),
    scratch_types=[
        pltpu.VMEM((2, 3 * B), jnp.float32),
        pltpu.VMEM((2, 8, B), jnp.int32),
        pltpu.VMEM((2, 8, B), jnp.float32),
        pltpu.VMEM((2, 8, B, FDIM), jnp.float32),
        pltpu.VMEM((2, B, FDIM), jnp.float32),
        pltpu.SemaphoreType.DMA,
        pltpu.SemaphoreType.DMA,
        pltpu.SemaphoreType.DMA,
    ],
    compiler_params=pltpu.CompilerParams(
        use_tc_tiling_on_sc=False, needs_layout_passes=False
    ),
)
def _grid_sample_sc(table, cq, out, cb, idxb, wb, rows, ob,
                    sem_c, sem_g, sem_o):
    _sc_body(table, cq, out, cb, idxb, wb, rows, ob, sem_c, sem_g, sem_o)


def kernel(x, feature):
    table = jnp.transpose(feature[0], (1, 2, 3, 0)).reshape(G * G * G, FDIM)

    def half(xh):
        cq = jnp.stack(
            [xh[:, 0].reshape(-1, B), xh[:, 1].reshape(-1, B),
             xh[:, 2].reshape(-1, B)], axis=1).reshape(-1)
        return _grid_sample_sc(table, cq)

    return jnp.concatenate(
        [half(x[i * H_PTS:(i + 1) * H_PTS]) for i in range(N_PTS // H_PTS)],
        axis=0)
